# Initial kernel scaffold; baseline (speedup 1.0000x reference)
import jax
import jax.numpy as jnp
from jax.experimental import pallas as pl


def kernel(x, laplacian_down, laplacian_up, weight):
    raise NotImplementedError("write your pallas kernel here")


# speedup vs baseline: 1.0339x; 1.0339x over previous
import functools

import jax
import jax.numpy as jnp
from jax.experimental import pallas as pl
from jax.experimental.pallas import tpu as pltpu

N = 4096
C = 32
BLOCK_ROWS = 512
NUM_BLOCKS = N // BLOCK_ROWS


def _snn_body(x_ref, ld_ref, lu_ref, w_ref, out_ref, p_buf, q_buf, yacc_buf):
    t = pl.program_id(0)
    i = pl.program_id(1)
    rows = pl.ds(i * BLOCK_ROWS, BLOCK_ROWS)
    w = w_ref[...]

    @pl.when(t == 0)
    def _pass1():
        xfull = x_ref[...]
        u1 = jnp.dot(ld_ref[...], xfull, preferred_element_type=jnp.float32)
        v1 = jnp.dot(lu_ref[...], xfull, preferred_element_type=jnp.float32)
        xi = x_ref[rows, :]
        yacc_buf[rows, :] = (
            jnp.dot(xi, w[:, :, 0], preferred_element_type=jnp.float32)
            + jnp.dot(u1, w[:, :, 1], preferred_element_type=jnp.float32)
            + jnp.dot(v1, w[:, :, 3], preferred_element_type=jnp.float32)
        )
        p_buf[rows, :] = jnp.dot(u1, w[:, :, 2], preferred_element_type=jnp.float32)
        q_buf[rows, :] = jnp.dot(v1, w[:, :, 4], preferred_element_type=jnp.float32)

    @pl.when(t == 1)
    def _pass2():
        y = (
            yacc_buf[rows, :]
            + jnp.dot(ld_ref[...], p_buf[...], preferred_element_type=jnp.float32)
            + jnp.dot(lu_ref[...], q_buf[...], preferred_element_type=jnp.float32)
        )
        out_ref[...] = jax.nn.sigmoid(y)


@jax.jit
def kernel(x, laplacian_down, laplacian_up, weight):
    return pl.pallas_call(
        _snn_body,
        grid=(2, NUM_BLOCKS),
        in_specs=[
            pl.BlockSpec((N, C), lambda t, i: (0, 0)),
            pl.BlockSpec((BLOCK_ROWS, N), lambda t, i: (i, 0)),
            pl.BlockSpec((BLOCK_ROWS, N), lambda t, i: (i, 0)),
            pl.BlockSpec(weight.shape, lambda t, i: (0, 0, 0)),
        ],
        out_specs=pl.BlockSpec((BLOCK_ROWS, C), lambda t, i: (i, 0)),
        out_shape=jax.ShapeDtypeStruct((N, C), jnp.float32),
        scratch_shapes=[
            pltpu.VMEM((N, C), jnp.float32),
            pltpu.VMEM((N, C), jnp.float32),
            pltpu.VMEM((N, C), jnp.float32),
        ],
        compiler_params=pltpu.CompilerParams(
            dimension_semantics=("arbitrary", "arbitrary"),
        ),
    )(x, laplacian_down, laplacian_up, weight)
